# Initial kernel scaffold; baseline (speedup 1.0000x reference)
import jax
import jax.numpy as jnp
from jax.experimental import pallas as pl


def kernel(z, edge_index):
    raise NotImplementedError("write your pallas kernel here")


# speedup vs baseline: 1.0914x; 1.0914x over previous
import jax
import jax.numpy as jnp
from jax import lax
from jax.experimental import pallas as pl
from jax.experimental.pallas import tpu as pltpu
from jax.experimental.pallas import tpu_sc as plsc

_D = 128
_L = 16
_NC = 2
_NS = 16
_NW = _NC * _NS
_E = 320000
_N = 10000
_EPW = _E // _NW
_C = 80
_NCHUNK = _EPW // _C
_NBUF = 2


def _sc_body(z_hbm, ei_hbm, out_hbm,
             sidx0, sidx1, didx0, didx1,
             srows0, srows1, drows0, drows1,
             outv0, outv1,
             ssem0, ssem1, dsem0, dsem1):
    sidx = (sidx0, sidx1)
    didx = (didx0, didx1)
    srows = (srows0, srows1)
    drows = (drows0, drows1)
    outv = (outv0, outv1)
    ssem = (ssem0, ssem1)
    dsem = (dsem0, dsem1)

    wid = lax.axis_index("s") * _NC + lax.axis_index("c")
    ebase = wid * _EPW

    def start(c, b):
        base = ebase + c * _C
        pltpu.sync_copy(ei_hbm.at[pl.ds(base, _C)], sidx[b])
        pltpu.sync_copy(ei_hbm.at[pl.ds(_E + base, _C)], didx[b])
        pltpu.async_copy(z_hbm.at[sidx[b]], srows[b], ssem[b])
        pltpu.async_copy(z_hbm.at[didx[b]], drows[b], dsem[b])

    def wait(b):
        pltpu.make_async_copy(z_hbm.at[sidx[b]], srows[b], ssem[b]).wait()
        pltpu.make_async_copy(z_hbm.at[didx[b]], drows[b], dsem[b]).wait()

    def compute_chunk(c, b):
        base = ebase + c * _C

        def g_body(g, carry):
            rows = g * _L + lax.iota(jnp.int32, _L)
            accs = [jnp.zeros((_L,), jnp.float32) for _ in range(4)]
            for d in range(_D):
                cols = jnp.full((_L,), d, jnp.int32)
                s = plsc.load_gather(srows[b], [rows, cols])
                t = plsc.load_gather(drows[b], [rows, cols])
                accs[d % 4] = accs[d % 4] + s * t
            acc = (accs[0] + accs[1]) + (accs[2] + accs[3])
            y = 1.0 / (1.0 + jnp.exp(-acc))
            outv[b][pl.ds(g * _L, _L)] = y
            return carry

        lax.fori_loop(0, _C // _L, g_body, 0)
        pltpu.sync_copy(outv[b], out_hbm.at[pl.ds(base, _C)])

    start(0, 0)

    def pair_body(i, carry):
        for b in range(_NBUF):
            c = _NBUF * i + b
            start(c + 1, 1 - b)
            wait(b)
            compute_chunk(c, b)
        return carry

    lax.fori_loop(0, (_NCHUNK - 1) // _NBUF, pair_body, 0)
    wait(0)
    compute_chunk(_NCHUNK - 1, 0)


def kernel(z, edge_index):
    ei = edge_index.astype(jnp.int32).reshape(-1)
    mesh = plsc.VectorSubcoreMesh(core_axis_name="c", subcore_axis_name="s")
    f = pl.kernel(
        _sc_body,
        out_type=jax.ShapeDtypeStruct((_E,), jnp.float32),
        mesh=mesh,
        compiler_params=pltpu.CompilerParams(needs_layout_passes=False),
        scratch_types=[
            pltpu.VMEM((_C,), jnp.int32),
            pltpu.VMEM((_C,), jnp.int32),
            pltpu.VMEM((_C,), jnp.int32),
            pltpu.VMEM((_C,), jnp.int32),
            pltpu.VMEM((_C, _D), jnp.float32),
            pltpu.VMEM((_C, _D), jnp.float32),
            pltpu.VMEM((_C, _D), jnp.float32),
            pltpu.VMEM((_C, _D), jnp.float32),
            pltpu.VMEM((_C,), jnp.float32),
            pltpu.VMEM((_C,), jnp.float32),
            pltpu.SemaphoreType.DMA,
            pltpu.SemaphoreType.DMA,
            pltpu.SemaphoreType.DMA,
            pltpu.SemaphoreType.DMA,
        ],
    )
    return f(z, ei)


# speedup vs baseline: 4.5195x; 4.1410x over previous
import jax
import jax.numpy as jnp
from jax import lax
from jax.experimental import pallas as pl
from jax.experimental.pallas import tpu as pltpu
from jax.experimental.pallas import tpu_sc as plsc

_D = 128
_K = _D // 16
_L = 16
_NC = 2
_NS = 16
_NW = _NC * _NS
_E = 320000
_EPW = _E // _NW
_C = 80
_NCHUNK = _EPW // _C
_NBUF = 2


def _sc_body(z_hbm, ei_hbm, out_hbm,
             sidx_all, didx_all,
             srows0, srows1, drows0, drows1,
             outv0, outv1,
             ssem0, ssem1, dsem0, dsem1, osem0, osem1):
    srows = (srows0, srows1)
    drows = (drows0, drows1)
    outv = (outv0, outv1)
    ssem = (ssem0, ssem1)
    dsem = (dsem0, dsem1)
    osem = (osem0, osem1)

    wid = lax.axis_index("s") * _NC + lax.axis_index("c")
    ebase = wid * _EPW

    pltpu.sync_copy(ei_hbm.at[pl.ds(ebase, _EPW)], sidx_all)
    pltpu.sync_copy(ei_hbm.at[pl.ds(_E + ebase, _EPW)], didx_all)

    lane_masks = [
        (lax.iota(jnp.int32, _L) == e) for e in range(_L)
    ]

    def start(c, b):
        off = c * _C
        pltpu.async_copy(z_hbm.at[sidx_all.at[pl.ds(off, _C)]], srows[b],
                         ssem[b])
        pltpu.async_copy(z_hbm.at[didx_all.at[pl.ds(off, _C)]], drows[b],
                         dsem[b])

    def wait_rows(c, b):
        off = c * _C
        pltpu.make_async_copy(z_hbm.at[sidx_all.at[pl.ds(off, _C)]],
                              srows[b], ssem[b]).wait()
        pltpu.make_async_copy(z_hbm.at[didx_all.at[pl.ds(off, _C)]],
                              drows[b], dsem[b]).wait()

    def wait_out(c, b):
        base = ebase + c * _C
        pltpu.make_async_copy(outv[b], out_hbm.at[pl.ds(base, _C)],
                              osem[b]).wait()

    def compute_chunk(c, b):
        base = ebase + c * _C

        def g_body(g, carry):
            y = jnp.zeros((_L,), jnp.float32)
            for e in range(_L):
                row = g * _L + e
                prods = []
                for k in range(_K):
                    s = srows[b][row, pl.ds(k * _L, _L)]
                    t = drows[b][row, pl.ds(k * _L, _L)]
                    prods.append(s * t)
                while len(prods) > 1:
                    prods = [prods[i] + prods[i + 1]
                             for i in range(0, len(prods), 2)]
                tot = jnp.sum(prods[0])
                y = jnp.where(lane_masks[e], tot, y)
            outv[b][pl.ds(g * _L, _L)] = 1.0 / (1.0 + jnp.exp(-y))
            return carry

        lax.fori_loop(0, _C // _L, g_body, 0)
        pltpu.async_copy(outv[b], out_hbm.at[pl.ds(base, _C)], osem[b])

    start(0, 0)

    def pair_body(i, carry):
        for b in range(_NBUF):
            c = _NBUF * i + b
            start(c + 1, 1 - b)
            wait_rows(c, b)

            @pl.when(c >= _NBUF)
            def _():
                wait_out(c - _NBUF, b)

            compute_chunk(c, b)
        return carry

    lax.fori_loop(0, (_NCHUNK - 1) // _NBUF, pair_body, 0)
    wait_rows(_NCHUNK - 1, 0)
    wait_out(_NCHUNK - 1 - _NBUF, 0)
    compute_chunk(_NCHUNK - 1, 0)
    wait_out(_NCHUNK - 2, 1)
    wait_out(_NCHUNK - 1, 0)


def kernel(z, edge_index):
    ei = edge_index.astype(jnp.int32).reshape(-1)
    mesh = plsc.VectorSubcoreMesh(core_axis_name="c", subcore_axis_name="s")
    f = pl.kernel(
        _sc_body,
        out_type=jax.ShapeDtypeStruct((_E,), jnp.float32),
        mesh=mesh,
        compiler_params=pltpu.CompilerParams(needs_layout_passes=False),
        scratch_types=[
            pltpu.VMEM((_EPW,), jnp.int32),
            pltpu.VMEM((_EPW,), jnp.int32),
            pltpu.VMEM((_C, _D), jnp.float32),
            pltpu.VMEM((_C, _D), jnp.float32),
            pltpu.VMEM((_C, _D), jnp.float32),
            pltpu.VMEM((_C, _D), jnp.float32),
            pltpu.VMEM((_C,), jnp.float32),
            pltpu.VMEM((_C,), jnp.float32),
            pltpu.SemaphoreType.DMA,
            pltpu.SemaphoreType.DMA,
            pltpu.SemaphoreType.DMA,
            pltpu.SemaphoreType.DMA,
            pltpu.SemaphoreType.DMA,
            pltpu.SemaphoreType.DMA,
        ],
    )
    return f(z, ei)
